# baseline (speedup 1.0000x reference)
import functools
import math

import jax
import jax.numpy as jnp
from jax import lax
from jax.experimental import pallas as pl
from jax.experimental.pallas import tpu as pltpu

N = 50000
E = 800000
EP = 819200
D = 128
H = 64
NG = 32
L = 3
CUT = 8.0
WID = 0.5

BN = 2000
BE = 3200
NBN = N // BN
NBE = EP // BE


def _softplus(x):
    return jnp.log(1.0 + jnp.exp(-jnp.abs(x))) + jnp.maximum(x, 0.0)


def _init_node_body(nf_ref, wne_ref, bne_ref, w1a_ref, w1b_ref,
                    nh_ref, a_ref, b_ref):
    nh = jnp.dot(nf_ref[...], wne_ref[...],
                 preferred_element_type=jnp.float32) + bne_ref[...]
    nh_ref[...] = nh
    a_ref[...] = jnp.dot(nh, w1a_ref[...], preferred_element_type=jnp.float32)
    b_ref[...] = jnp.dot(nh, w1b_ref[...], preferred_element_type=jnp.float32)


def _init_node(node_features, Wne, bne, W1a, W1b):
    f32 = jnp.float32
    return pl.pallas_call(
        _init_node_body,
        grid=(NBN,),
        in_specs=[
            pl.BlockSpec((BN, D), lambda i: (i, 0)),
            pl.BlockSpec((D, H), lambda i: (0, 0)),
            pl.BlockSpec((1, H), lambda i: (0, 0)),
            pl.BlockSpec((H, H), lambda i: (0, 0)),
            pl.BlockSpec((H, H), lambda i: (0, 0)),
        ],
        out_specs=[
            pl.BlockSpec((BN, H), lambda i: (i, 0)),
            pl.BlockSpec((BN, H), lambda i: (i, 0)),
            pl.BlockSpec((BN, H), lambda i: (i, 0)),
        ],
        out_shape=[jax.ShapeDtypeStruct((N, H), f32)] * 3,
    )(node_features, Wne, bne.reshape(1, H), W1a, W1b)


def _edge_body(first, g_ref, eh_ref, eb3_ref, gtab_ref, w1c_ref, w2_ref,
               b2_ref, cen_ref, wee_ref, bee_ref, out_ref):
    pid = pl.program_id(0)
    if first:
        a = eh_ref[0, 0, :]
        diff = a[:, None] - cen_ref[...]
        rbf = jnp.exp(-0.5 * (diff / WID) ** 2)
        cutf = 0.5 * (jnp.cos((math.pi / CUT) * a) + 1.0)
        cutf = cutf * (a < CUT).astype(jnp.float32)
        eh = jnp.dot(rbf * cutf[:, None], wee_ref[...],
                     preferred_element_type=jnp.float32) + bee_ref[...]
    else:
        eh = eh_ref[...]
    ebv = eb3_ref[0, 0, :]
    oh = (ebv[:, None] == lax.broadcasted_iota(jnp.int32, (1, NG), 1)
          ).astype(jnp.float32)
    pre = (g_ref[...]
           + jnp.dot(eh, w1c_ref[...], preferred_element_type=jnp.float32)
           + jnp.dot(oh, gtab_ref[...], preferred_element_type=jnp.float32))
    out = eh + jnp.dot(_softplus(pre), w2_ref[...],
                       preferred_element_type=jnp.float32) + b2_ref[...]
    eidx = pid * BE + lax.broadcasted_iota(jnp.int32, (BE, 1), 0)
    out_ref[...] = jnp.where(eidx < E, out, 0.0)


def _edge_layer(first, G, eh_or_attr3, eb3, gtab, W1c, W2, b2, centers, Wee, bee):
    f32 = jnp.float32
    eh_spec = (pl.BlockSpec((1, 1, BE), lambda i: (i, 0, 0)) if first
               else pl.BlockSpec((BE, H), lambda i: (i, 0)))
    return pl.pallas_call(
        functools.partial(_edge_body, first),
        grid=(NBE,),
        in_specs=[
            pl.BlockSpec((BE, H), lambda i: (i, 0)),
            eh_spec,
            pl.BlockSpec((1, 1, BE), lambda i: (i, 0, 0)),
            pl.BlockSpec((NG, H), lambda i: (0, 0)),
            pl.BlockSpec((H, H), lambda i: (0, 0)),
            pl.BlockSpec((H, H), lambda i: (0, 0)),
            pl.BlockSpec((1, H), lambda i: (0, 0)),
            pl.BlockSpec((1, H), lambda i: (0, 0)),
            pl.BlockSpec((H, H), lambda i: (0, 0)),
            pl.BlockSpec((1, H), lambda i: (0, 0)),
        ],
        out_specs=pl.BlockSpec((BE, H), lambda i: (i, 0)),
        out_shape=jax.ShapeDtypeStruct((EP, H), f32),
    )(G, eh_or_attr3, eb3, gtab, W1c, W2, b2,
      centers.reshape(1, H), Wee, bee.reshape(1, H))


def _node_body(nh_ref, mg_ref, b3_ref, gtab_ref, n1a_ref, n1b_ref, w2_ref,
               b2_ref, w1a_ref, w1b_ref,
               nn_ref, a_ref, bt_ref, gn_ref, ge_ref):
    pid = pl.program_id(0)
    nh = nh_ref[...]
    mg = mg_ref[...]
    bv = b3_ref[0, 0, :]
    oh = (bv[:, None] == lax.broadcasted_iota(jnp.int32, (1, NG), 1)
          ).astype(jnp.float32)
    pre = (jnp.dot(nh, n1a_ref[...], preferred_element_type=jnp.float32)
           + jnp.dot(mg, n1b_ref[...], preferred_element_type=jnp.float32)
           + jnp.dot(oh, gtab_ref[...], preferred_element_type=jnp.float32))
    nn = nh + jnp.dot(_softplus(pre), w2_ref[...],
                      preferred_element_type=jnp.float32) + b2_ref[...]
    nn_ref[...] = nn
    a_ref[...] = jnp.dot(nn, w1a_ref[...], preferred_element_type=jnp.float32)
    bt_ref[...] = jnp.dot(nn, w1b_ref[...], preferred_element_type=jnp.float32)
    dn = (((0,), (0,)), ((), ()))
    gn_p = lax.dot_general(oh, nn, dn, preferred_element_type=jnp.float32)
    ge_p = lax.dot_general(oh, mg, dn, preferred_element_type=jnp.float32)

    @pl.when(pid == 0)
    def _():
        gn_ref[...] = gn_p
        ge_ref[...] = ge_p

    @pl.when(pid != 0)
    def _():
        gn_ref[...] += gn_p
        ge_ref[...] += ge_p


def _node_layer(node_h, msg, b3, gtab, N1a, N1b, W2, b2, W1a_nxt, W1b_nxt):
    f32 = jnp.float32
    return pl.pallas_call(
        _node_body,
        grid=(NBN,),
        in_specs=[
            pl.BlockSpec((BN, H), lambda i: (i, 0)),
            pl.BlockSpec((BN, H), lambda i: (i, 0)),
            pl.BlockSpec((1, 1, BN), lambda i: (i, 0, 0)),
            pl.BlockSpec((NG, H), lambda i: (0, 0)),
            pl.BlockSpec((H, H), lambda i: (0, 0)),
            pl.BlockSpec((H, H), lambda i: (0, 0)),
            pl.BlockSpec((H, H), lambda i: (0, 0)),
            pl.BlockSpec((1, H), lambda i: (0, 0)),
            pl.BlockSpec((H, H), lambda i: (0, 0)),
            pl.BlockSpec((H, H), lambda i: (0, 0)),
        ],
        out_specs=[
            pl.BlockSpec((BN, H), lambda i: (i, 0)),
            pl.BlockSpec((BN, H), lambda i: (i, 0)),
            pl.BlockSpec((BN, H), lambda i: (i, 0)),
            pl.BlockSpec((NG, H), lambda i: (0, 0)),
            pl.BlockSpec((NG, H), lambda i: (0, 0)),
        ],
        out_shape=[
            jax.ShapeDtypeStruct((N, H), f32),
            jax.ShapeDtypeStruct((N, H), f32),
            jax.ShapeDtypeStruct((N, H), f32),
            jax.ShapeDtypeStruct((NG, H), f32),
            jax.ShapeDtypeStruct((NG, H), f32),
        ],
    )(node_h, msg, b3, gtab, N1a, N1b, W2, b2, W1a_nxt, W1b_nxt)


def _gather_G(A, B, row_p, col_p):
    return jnp.take(A, row_p, axis=0) + jnp.take(B, col_p, axis=0)


def _scatter_msg(edge_new, row_p):
    return jax.ops.segment_sum(edge_new, row_p, num_segments=N)


def kernel(node_features, edge_index, edge_attr, batch, Wne, bne, Wee, bee,
           Wge, bge, eW1, eb1, eW2, eb2, nW1, nb1, nW2, nb2,
           gW1, gb1, gW2, gb2, oW1, ob1, oW2, ob2, centers):
    f32 = jnp.float32
    sp = jax.nn.softplus
    row = edge_index[0]
    col = edge_index[1]
    pad = EP - E
    row_p = jnp.pad(row, (0, pad))
    col_p = jnp.pad(col, (0, pad))
    attr_p = jnp.pad(edge_attr, (0, pad))
    edge_batch = batch[row]
    eb_p = jnp.pad(edge_batch, (0, pad))
    eb3 = eb_p.reshape(NBE, 1, BE)
    attr3 = attr_p.reshape(NBE, 1, BE)
    b3 = batch.reshape(NBN, 1, BN)

    cnt_n = jnp.bincount(batch, length=NG).astype(f32)
    cnt_e = jnp.bincount(edge_batch, length=NG).astype(f32)

    global_h = jnp.ones((NG, 1), f32) @ Wge + bge

    eW1a = eW1[:, 0:H, :]
    eW1b = eW1[:, H:2 * H, :]
    eW1c = eW1[:, 2 * H:3 * H, :]
    eW1d = eW1[:, 3 * H:, :]
    nW1a = nW1[:, 0:H, :]
    nW1b = nW1[:, H:2 * H, :]
    nW1c = nW1[:, 2 * H:, :]
    gW1a = gW1[:, 0:H, :]
    gW1b = gW1[:, H:2 * H, :]
    gW1c = gW1[:, 2 * H:, :]

    node_h, A, B = _init_node(node_features, Wne, bne, eW1a[0], eW1b[0])

    eh_or_attr3 = attr3
    zW = jnp.zeros((H, H), f32)
    for i in range(L):
        gtab_e = global_h @ eW1d[i] + eb1[i]
        G = _gather_G(A, B, row_p, col_p)
        edge_new = _edge_layer(i == 0, G, eh_or_attr3, eb3, gtab_e,
                               eW1c[i], eW2[i], eb2[i].reshape(1, H),
                               centers, Wee, bee)
        msg = _scatter_msg(edge_new, row_p)
        gtab_n = global_h @ nW1c[i] + nb1[i]
        nxt_a = eW1a[i + 1] if i + 1 < L else zW
        nxt_b = eW1b[i + 1] if i + 1 < L else zW
        node_h, A, B, gn_sum, ge_sum = _node_layer(
            node_h, msg, b3, gtab_n, nW1a[i], nW1b[i], nW2[i],
            nb2[i].reshape(1, H), nxt_a, nxt_b)
        gn = gn_sum / cnt_n[:, None]
        ge = ge_sum / cnt_e[:, None]
        pre_g = gn @ gW1a[i] + ge @ gW1b[i] + global_h @ gW1c[i] + gb1[i]
        global_h = global_h + sp(pre_g) @ gW2[i] + gb2[i]
        eh_or_attr3 = edge_new

    return sp(global_h @ oW1 + ob1) @ oW2 + ob2


# speedup vs baseline: 2.9227x; 2.9227x over previous
import functools
import math

import jax
import jax.numpy as jnp
from jax import lax
from jax.experimental import pallas as pl
from jax.experimental.pallas import tpu as pltpu
from jax.experimental.pallas import tpu_sc as plsc

N = 50000
E = 800000
EP = 819200
D = 128
H = 64
NG = 32
L = 3
CUT = 8.0
WID = 0.5

BN = 2000
BE = 3200
NBN = N // BN
NBE = EP // BE


def _softplus(x):
    return jnp.log(1.0 + jnp.exp(-jnp.abs(x))) + jnp.maximum(x, 0.0)


def _init_node_body(nf_ref, wne_ref, bne_ref, w1a_ref, w1b_ref,
                    nh_ref, a_ref, b_ref):
    nh = jnp.dot(nf_ref[...], wne_ref[...],
                 preferred_element_type=jnp.float32) + bne_ref[...]
    nh_ref[...] = nh
    a_ref[...] = jnp.dot(nh, w1a_ref[...], preferred_element_type=jnp.float32)
    b_ref[...] = jnp.dot(nh, w1b_ref[...], preferred_element_type=jnp.float32)


def _init_node(node_features, Wne, bne, W1a, W1b):
    f32 = jnp.float32
    return pl.pallas_call(
        _init_node_body,
        grid=(NBN,),
        in_specs=[
            pl.BlockSpec((BN, D), lambda i: (i, 0)),
            pl.BlockSpec((D, H), lambda i: (0, 0)),
            pl.BlockSpec((1, H), lambda i: (0, 0)),
            pl.BlockSpec((H, H), lambda i: (0, 0)),
            pl.BlockSpec((H, H), lambda i: (0, 0)),
        ],
        out_specs=[
            pl.BlockSpec((BN, H), lambda i: (i, 0)),
            pl.BlockSpec((BN, H), lambda i: (i, 0)),
            pl.BlockSpec((BN, H), lambda i: (i, 0)),
        ],
        out_shape=[jax.ShapeDtypeStruct((N, H), f32)] * 3,
    )(node_features, Wne, bne.reshape(1, H), W1a, W1b)


def _edge_body(first, g_ref, eh_ref, eb3_ref, gtab_ref, w1c_ref, w2_ref,
               b2_ref, cen_ref, wee_ref, bee_ref, out_ref):
    pid = pl.program_id(0)
    if first:
        a = eh_ref[0, 0, :]
        diff = a[:, None] - cen_ref[...]
        rbf = jnp.exp(-0.5 * (diff / WID) ** 2)
        cutf = 0.5 * (jnp.cos((math.pi / CUT) * a) + 1.0)
        cutf = cutf * (a < CUT).astype(jnp.float32)
        eh = jnp.dot(rbf * cutf[:, None], wee_ref[...],
                     preferred_element_type=jnp.float32) + bee_ref[...]
    else:
        eh = eh_ref[...]
    ebv = eb3_ref[0, 0, :]
    oh = (ebv[:, None] == lax.broadcasted_iota(jnp.int32, (1, NG), 1)
          ).astype(jnp.float32)
    pre = (g_ref[...]
           + jnp.dot(eh, w1c_ref[...], preferred_element_type=jnp.float32)
           + jnp.dot(oh, gtab_ref[...], preferred_element_type=jnp.float32))
    out = eh + jnp.dot(_softplus(pre), w2_ref[...],
                       preferred_element_type=jnp.float32) + b2_ref[...]
    eidx = pid * BE + lax.broadcasted_iota(jnp.int32, (BE, 1), 0)
    out_ref[...] = jnp.where(eidx < E, out, 0.0)


def _edge_layer(first, G, eh_or_attr3, eb3, gtab, W1c, W2, b2, centers, Wee, bee):
    f32 = jnp.float32
    eh_spec = (pl.BlockSpec((1, 1, BE), lambda i: (i, 0, 0)) if first
               else pl.BlockSpec((BE, H), lambda i: (i, 0)))
    return pl.pallas_call(
        functools.partial(_edge_body, first),
        grid=(NBE,),
        in_specs=[
            pl.BlockSpec((BE, H), lambda i: (i, 0)),
            eh_spec,
            pl.BlockSpec((1, 1, BE), lambda i: (i, 0, 0)),
            pl.BlockSpec((NG, H), lambda i: (0, 0)),
            pl.BlockSpec((H, H), lambda i: (0, 0)),
            pl.BlockSpec((H, H), lambda i: (0, 0)),
            pl.BlockSpec((1, H), lambda i: (0, 0)),
            pl.BlockSpec((1, H), lambda i: (0, 0)),
            pl.BlockSpec((H, H), lambda i: (0, 0)),
            pl.BlockSpec((1, H), lambda i: (0, 0)),
        ],
        out_specs=pl.BlockSpec((BE, H), lambda i: (i, 0)),
        out_shape=jax.ShapeDtypeStruct((EP, H), f32),
    )(G, eh_or_attr3, eb3, gtab, W1c, W2, b2,
      centers.reshape(1, H), Wee, bee.reshape(1, H))


def _node_body(nh_ref, mg_ref, b3_ref, gtab_ref, n1a_ref, n1b_ref, w2_ref,
               b2_ref, w1a_ref, w1b_ref,
               nn_ref, a_ref, bt_ref, gn_ref, ge_ref):
    pid = pl.program_id(0)
    nh = nh_ref[...]
    mg = mg_ref[...]
    bv = b3_ref[0, 0, :]
    oh = (bv[:, None] == lax.broadcasted_iota(jnp.int32, (1, NG), 1)
          ).astype(jnp.float32)
    pre = (jnp.dot(nh, n1a_ref[...], preferred_element_type=jnp.float32)
           + jnp.dot(mg, n1b_ref[...], preferred_element_type=jnp.float32)
           + jnp.dot(oh, gtab_ref[...], preferred_element_type=jnp.float32))
    nn = nh + jnp.dot(_softplus(pre), w2_ref[...],
                      preferred_element_type=jnp.float32) + b2_ref[...]
    nn_ref[...] = nn
    a_ref[...] = jnp.dot(nn, w1a_ref[...], preferred_element_type=jnp.float32)
    bt_ref[...] = jnp.dot(nn, w1b_ref[...], preferred_element_type=jnp.float32)
    dn = (((0,), (0,)), ((), ()))
    gn_p = lax.dot_general(oh, nn, dn, preferred_element_type=jnp.float32)
    ge_p = lax.dot_general(oh, mg, dn, preferred_element_type=jnp.float32)

    @pl.when(pid == 0)
    def _():
        gn_ref[...] = gn_p
        ge_ref[...] = ge_p

    @pl.when(pid != 0)
    def _():
        gn_ref[...] += gn_p
        ge_ref[...] += ge_p


def _node_layer(node_h, msg, b3, gtab, N1a, N1b, W2, b2, W1a_nxt, W1b_nxt):
    f32 = jnp.float32
    return pl.pallas_call(
        _node_body,
        grid=(NBN,),
        in_specs=[
            pl.BlockSpec((BN, H), lambda i: (i, 0)),
            pl.BlockSpec((BN, H), lambda i: (i, 0)),
            pl.BlockSpec((1, 1, BN), lambda i: (i, 0, 0)),
            pl.BlockSpec((NG, H), lambda i: (0, 0)),
            pl.BlockSpec((H, H), lambda i: (0, 0)),
            pl.BlockSpec((H, H), lambda i: (0, 0)),
            pl.BlockSpec((H, H), lambda i: (0, 0)),
            pl.BlockSpec((1, H), lambda i: (0, 0)),
            pl.BlockSpec((H, H), lambda i: (0, 0)),
            pl.BlockSpec((H, H), lambda i: (0, 0)),
        ],
        out_specs=[
            pl.BlockSpec((BN, H), lambda i: (i, 0)),
            pl.BlockSpec((BN, H), lambda i: (i, 0)),
            pl.BlockSpec((BN, H), lambda i: (i, 0)),
            pl.BlockSpec((NG, H), lambda i: (0, 0)),
            pl.BlockSpec((NG, H), lambda i: (0, 0)),
        ],
        out_shape=[
            jax.ShapeDtypeStruct((N, H), f32),
            jax.ShapeDtypeStruct((N, H), f32),
            jax.ShapeDtypeStruct((N, H), f32),
            jax.ShapeDtypeStruct((NG, H), f32),
            jax.ShapeDtypeStruct((NG, H), f32),
        ],
    )(node_h, msg, b3, gtab, N1a, N1b, W2, b2, W1a_nxt, W1b_nxt)


NC = 2
NS = 16
NW = NC * NS
EW = EP // NW
CH = 512
NCH = EW // CH
_SC_MESH = plsc.VectorSubcoreMesh(core_axis_name="c", subcore_axis_name="s")


def _gather_body(a_hbm, b_hbm, row4, col4, g_hbm,
                 idxr, idxc, bufA, bufB, semA, semB):
    cid = lax.axis_index("c")
    sid = lax.axis_index("s")
    wid = sid * NC + cid
    for c in range(NCH):
        base = wid * EW + c * CH
        b4 = wid * (EW // 128) + c * (CH // 128)
        pltpu.sync_copy(row4.at[pl.ds(b4, CH // 128)], idxr)
        pltpu.sync_copy(col4.at[pl.ds(b4, CH // 128)], idxc)
        cps = []
        for j in range(CH // 128):
            cps.append(pltpu.async_copy(
                a_hbm.at[idxr.at[j]], bufA.at[pl.ds(j * 128, 128)], semA))
            cps.append(pltpu.async_copy(
                b_hbm.at[idxc.at[j]], bufB.at[pl.ds(j * 128, 128)], semB))
        for cp in cps:
            cp.wait()

        def addbody(r, carry):
            for k in range(H // 16):
                s = 16 * k
                bufA[r, pl.ds(s, 16)] = bufA[r, pl.ds(s, 16)] + bufB[r, pl.ds(s, 16)]
            return carry

        lax.fori_loop(0, CH, addbody, 0)
        pltpu.sync_copy(bufA, g_hbm.at[pl.ds(base, CH)])


def _gather_G(A, B, row4, col4):
    f32 = jnp.float32
    return pl.kernel(
        _gather_body,
        out_type=jax.ShapeDtypeStruct((EP, H), f32),
        mesh=_SC_MESH,
        scratch_types=[
            pltpu.VMEM((CH // 128, 128), jnp.int32),
            pltpu.VMEM((CH // 128, 128), jnp.int32),
            pltpu.VMEM((CH, H), f32),
            pltpu.VMEM((CH, H), f32),
            pltpu.SemaphoreType.DMA,
            pltpu.SemaphoreType.DMA,
        ],
        compiler_params=pltpu.CompilerParams(use_tc_tiling_on_sc=False),
    )(A, B, row4, col4)


def _scatter_msg(edge_new, row_p):
    return jax.ops.segment_sum(edge_new, row_p, num_segments=N)


def kernel(node_features, edge_index, edge_attr, batch, Wne, bne, Wee, bee,
           Wge, bge, eW1, eb1, eW2, eb2, nW1, nb1, nW2, nb2,
           gW1, gb1, gW2, gb2, oW1, ob1, oW2, ob2, centers):
    f32 = jnp.float32
    sp = jax.nn.softplus
    row = edge_index[0]
    col = edge_index[1]
    pad = EP - E
    row_p = jnp.pad(row, (0, pad))
    col_p = jnp.pad(col, (0, pad))
    row4 = row_p.reshape(EP // 128, 128)
    col4 = col_p.reshape(EP // 128, 128)
    attr_p = jnp.pad(edge_attr, (0, pad))
    bnd = jnp.sum(batch[:, None] < jnp.arange(NG + 1)[None, :], axis=0)
    edge_batch = (jnp.sum(row[:, None] >= bnd[None, :NG], axis=1) - 1
                  ).astype(jnp.int32)
    eb_p = jnp.pad(edge_batch, (0, pad))
    eb3 = eb_p.reshape(NBE, 1, BE)
    attr3 = attr_p.reshape(NBE, 1, BE)
    b3 = batch.reshape(NBN, 1, BN)

    cnt_n = (bnd[1:] - bnd[:NG]).astype(f32)
    cnt_e = jnp.sum(
        (edge_batch[:, None] == jnp.arange(NG)[None, :]).astype(f32), axis=0)

    global_h = jnp.ones((NG, 1), f32) @ Wge + bge

    eW1a = eW1[:, 0:H, :]
    eW1b = eW1[:, H:2 * H, :]
    eW1c = eW1[:, 2 * H:3 * H, :]
    eW1d = eW1[:, 3 * H:, :]
    nW1a = nW1[:, 0:H, :]
    nW1b = nW1[:, H:2 * H, :]
    nW1c = nW1[:, 2 * H:, :]
    gW1a = gW1[:, 0:H, :]
    gW1b = gW1[:, H:2 * H, :]
    gW1c = gW1[:, 2 * H:, :]

    node_h, A, B = _init_node(node_features, Wne, bne, eW1a[0], eW1b[0])

    eh_or_attr3 = attr3
    zW = jnp.zeros((H, H), f32)
    for i in range(L):
        gtab_e = global_h @ eW1d[i] + eb1[i]
        G = _gather_G(A, B, row4, col4)
        edge_new = _edge_layer(i == 0, G, eh_or_attr3, eb3, gtab_e,
                               eW1c[i], eW2[i], eb2[i].reshape(1, H),
                               centers, Wee, bee)
        msg = _scatter_msg(edge_new, row_p)
        gtab_n = global_h @ nW1c[i] + nb1[i]
        nxt_a = eW1a[i + 1] if i + 1 < L else zW
        nxt_b = eW1b[i + 1] if i + 1 < L else zW
        node_h, A, B, gn_sum, ge_sum = _node_layer(
            node_h, msg, b3, gtab_n, nW1a[i], nW1b[i], nW2[i],
            nb2[i].reshape(1, H), nxt_a, nxt_b)
        gn = gn_sum / cnt_n[:, None]
        ge = ge_sum / cnt_e[:, None]
        pre_g = gn @ gW1a[i] + ge @ gW1b[i] + global_h @ gW1c[i] + gb1[i]
        global_h = global_h + sp(pre_g) @ gW2[i] + gb2[i]
        eh_or_attr3 = edge_new

    return sp(global_h @ oW1 + ob1) @ oW2 + ob2
